# baseline (speedup 1.0000x reference)
import functools

import jax
import jax.numpy as jnp
from jax import lax
from jax.experimental import pallas as pl
from jax.experimental.pallas import tpu as pltpu
from jax.experimental.pallas import tpu_sc as plsc

_EPS = 1e-08
_N = 3276800
_NC = 2
_NS = 16
_NW = _NC * _NS
_NBUF = 4
_CHUNK = 6400
_L = 16

_E_SC = 1638400
_PER_W = _E_SC // _NW
_NCHUNK = _PER_W // _CHUNK

_COLS = 1024
_ROWS = _N // _COLS
_R_SC = _E_SC // _COLS
_BR = 320


def _sc_body(x_hbm, g_hbm, mins_hbm, maxs_hbm, out_hbm, scale_v, offs_v, *rest):
    xbufs = rest[0:_NBUF]
    gbufs = rest[_NBUF:2 * _NBUF]
    obufs = rest[2 * _NBUF:3 * _NBUF]
    ld_sems = rest[3 * _NBUF:4 * _NBUF]
    st_sems = rest[4 * _NBUF:5 * _NBUF]

    wid = lax.axis_index("s") * _NC + lax.axis_index("c")
    base = wid * _PER_W

    def start_load(c, b):
        off = base + c * _CHUNK
        pltpu.async_copy(x_hbm.at[pl.ds(off, _CHUNK)], xbufs[b], ld_sems[b])
        pltpu.async_copy(g_hbm.at[pl.ds(off, _CHUNK)], gbufs[b], ld_sems[b])

    def wait_load(b):
        pltpu.make_async_copy(
            x_hbm.at[pl.ds(0, _CHUNK)], xbufs[b], ld_sems[b]).wait()
        pltpu.make_async_copy(
            g_hbm.at[pl.ds(0, _CHUNK)], gbufs[b], ld_sems[b]).wait()

    def wait_store(b):
        pltpu.make_async_copy(
            obufs[b], out_hbm.at[pl.ds(0, _CHUNK)], st_sems[b]).wait()

    def start_store(c, b):
        off = base + c * _CHUNK
        pltpu.async_copy(obufs[b], out_hbm.at[pl.ds(off, _CHUNK)], st_sems[b])

    def compute(b, scale_reg, offs_reg):
        xb, gb, ob = xbufs[b], gbufs[b], obufs[b]

        @plsc.parallel_loop(0, _CHUNK // _L, unroll=8)
        def vec_body(i):
            s = pl.ds(i * _L, _L)
            idx = gb[s] - 1
            sg = jnp.take_along_axis(
                scale_reg, idx, axis=0, mode="promise_in_bounds")
            og = jnp.take_along_axis(
                offs_reg, idx, axis=0, mode="promise_in_bounds")
            ob[s] = xb[s] * sg + og

    for b in range(_NBUF - 1):
        start_load(b, b)

    pltpu.sync_copy(mins_hbm, scale_v)
    pltpu.sync_copy(maxs_hbm, offs_v)
    m = scale_v[...]
    M = offs_v[...]
    scale_reg = (1.0 - 2.0 * _EPS) / (M - m)
    offs_reg = _EPS - m * scale_reg

    @pl.loop(0, _NCHUNK, step=_NBUF)
    def chunk_ring(c):
        for j in range(_NBUF):
            b = j
            cj = c + j

            wait_load(b)

            @pl.when(cj >= _NBUF)
            def _():
                wait_store(b)

            compute(b, scale_reg, offs_reg)
            start_store(cj, b)

            @pl.when(cj + _NBUF - 1 < _NCHUNK)
            def _():
                start_load(cj + _NBUF - 1, (b + _NBUF - 1) % _NBUF)

    for b in range(_NBUF):
        wait_store(b)


def _tc_body(x_ref, g_ref, mins_ref, maxs_ref, o_ref):
    m = mins_ref[...]
    M = maxs_ref[...]
    sc16 = (1.0 - 2.0 * _EPS) / (M - m)
    of16 = _EPS - m * sc16
    idx = g_ref[...] - 1
    tab_s = jnp.broadcast_to(sc16[None, :], (_BR, 16))
    tab_o = jnp.broadcast_to(of16[None, :], (_BR, 16))
    sg = jnp.take_along_axis(tab_s, idx, axis=1, mode="promise_in_bounds")
    og = jnp.take_along_axis(tab_o, idx, axis=1, mode="promise_in_bounds")
    o_ref[...] = x_ref[...] * sg + og


@jax.jit
def _run(x, group, mins, maxs):
    mesh = plsc.VectorSubcoreMesh(core_axis_name="c", subcore_axis_name="s")
    sc_kern = functools.partial(
        pl.kernel,
        mesh=mesh,
        compiler_params=pltpu.CompilerParams(needs_layout_passes=False),
        out_type=jax.ShapeDtypeStruct((_N,), jnp.float32),
        scratch_types=(
            [
                pltpu.VMEM((_L,), jnp.float32),
                pltpu.VMEM((_L,), jnp.float32),
            ]
            + [pltpu.VMEM((_CHUNK,), jnp.float32) for _ in range(_NBUF)]
            + [pltpu.VMEM((_CHUNK,), jnp.int32) for _ in range(_NBUF)]
            + [pltpu.VMEM((_CHUNK,), jnp.float32) for _ in range(_NBUF)]
            + [pltpu.SemaphoreType.DMA for _ in range(_NBUF)]
            + [pltpu.SemaphoreType.DMA for _ in range(_NBUF)]
        ),
    )(_sc_body)
    sc_out = sc_kern(x, group, mins, maxs)

    x2 = x.reshape(_ROWS, _COLS)
    g2 = group.reshape(_ROWS, _COLS)
    n_tc_blocks = (_ROWS - _R_SC) // _BR
    tc_out = pl.pallas_call(
        _tc_body,
        grid=(n_tc_blocks,),
        in_specs=[
            pl.BlockSpec((_BR, _COLS), lambda i: (_R_SC // _BR + i, 0)),
            pl.BlockSpec((_BR, _COLS), lambda i: (_R_SC // _BR + i, 0)),
            pl.BlockSpec((_L,), lambda i: (0,)),
            pl.BlockSpec((_L,), lambda i: (0,)),
        ],
        out_specs=pl.BlockSpec((_BR, _COLS), lambda i: (i, 0)),
        out_shape=jax.ShapeDtypeStruct((_ROWS - _R_SC, _COLS), jnp.float32),
        compiler_params=pltpu.CompilerParams(
            dimension_semantics=("arbitrary",)),
    )(x2, g2, mins, maxs)

    return lax.dynamic_update_slice(sc_out, tc_out.reshape(-1), (_E_SC,))


def kernel(x, group, mins, maxs):
    return _run(x, group, mins, maxs)


# speedup vs baseline: 2.1695x; 2.1695x over previous
import functools

import jax
import jax.numpy as jnp
from jax import lax
from jax.experimental import pallas as pl
from jax.experimental.pallas import tpu as pltpu
from jax.experimental.pallas import tpu_sc as plsc

_EPS = 1e-08
_N = 3276800
_NC = 2
_NS = 16
_NW = _NC * _NS
_NBUF = 4
_CHUNK = 6400
_L = 16

_E_SC = 1638400
_PER_W = _E_SC // _NW
_NCHUNK = _PER_W // _CHUNK

_COLS = 1024
_BLK = 204800
_BR = _BLK // _COLS
_N_TC_BLOCKS = (_N - _E_SC) // _BLK


def _sc_body(x_hbm, g_hbm, mins_hbm, maxs_hbm, out_hbm, scale_v, offs_v, *rest):
    xbufs = rest[0:_NBUF]
    gbufs = rest[_NBUF:2 * _NBUF]
    obufs = rest[2 * _NBUF:3 * _NBUF]
    ld_sems = rest[3 * _NBUF:4 * _NBUF]
    st_sems = rest[4 * _NBUF:5 * _NBUF]

    wid = lax.axis_index("s") * _NC + lax.axis_index("c")
    base = wid * _PER_W

    def start_load(c, b):
        off = base + c * _CHUNK
        pltpu.async_copy(x_hbm.at[pl.ds(off, _CHUNK)], xbufs[b], ld_sems[b])
        pltpu.async_copy(g_hbm.at[pl.ds(off, _CHUNK)], gbufs[b], ld_sems[b])

    def wait_load(b):
        pltpu.make_async_copy(
            x_hbm.at[pl.ds(0, _CHUNK)], xbufs[b], ld_sems[b]).wait()
        pltpu.make_async_copy(
            g_hbm.at[pl.ds(0, _CHUNK)], gbufs[b], ld_sems[b]).wait()

    def wait_store(b):
        pltpu.make_async_copy(
            obufs[b], out_hbm.at[pl.ds(0, _CHUNK)], st_sems[b]).wait()

    def start_store(c, b):
        off = base + c * _CHUNK
        pltpu.async_copy(obufs[b], out_hbm.at[pl.ds(off, _CHUNK)], st_sems[b])

    def compute(b, scale_reg, offs_reg):
        xb, gb, ob = xbufs[b], gbufs[b], obufs[b]

        @plsc.parallel_loop(0, _CHUNK // _L, unroll=8)
        def vec_body(i):
            s = pl.ds(i * _L, _L)
            idx = gb[s] - 1
            sg = jnp.take_along_axis(
                scale_reg, idx, axis=0, mode="promise_in_bounds")
            og = jnp.take_along_axis(
                offs_reg, idx, axis=0, mode="promise_in_bounds")
            ob[s] = xb[s] * sg + og

    for b in range(_NBUF - 1):
        start_load(b, b)

    pltpu.sync_copy(mins_hbm, scale_v)
    pltpu.sync_copy(maxs_hbm, offs_v)
    m = scale_v[...]
    M = offs_v[...]
    scale_reg = (1.0 - 2.0 * _EPS) / (M - m)
    offs_reg = _EPS - m * scale_reg

    @pl.loop(0, _NCHUNK, step=_NBUF)
    def chunk_ring(c):
        for j in range(_NBUF):
            b = j
            cj = c + j

            wait_load(b)

            @pl.when(cj >= _NBUF)
            def _():
                wait_store(b)

            compute(b, scale_reg, offs_reg)
            start_store(cj, b)

            @pl.when(cj + _NBUF - 1 < _NCHUNK)
            def _():
                start_load(cj + _NBUF - 1, (b + _NBUF - 1) % _NBUF)

    for b in range(_NBUF):
        wait_store(b)


def _tc_body(x_ref, g_ref, mins_ref, maxs_ref, o_ref):
    m = mins_ref[...]
    M = maxs_ref[...]
    sc16 = (1.0 - 2.0 * _EPS) / (M - m)
    of16 = _EPS - m * sc16
    idx = g_ref[...].reshape(_BR, _COLS) - 1
    tab_s = jnp.broadcast_to(sc16[None, :], (_BR, 16))
    tab_o = jnp.broadcast_to(of16[None, :], (_BR, 16))
    sg = jnp.take_along_axis(tab_s, idx, axis=1, mode="promise_in_bounds")
    og = jnp.take_along_axis(tab_o, idx, axis=1, mode="promise_in_bounds")
    res = x_ref[...].reshape(_BR, _COLS) * sg + og
    o_ref[...] = res.reshape(_BLK)


@jax.jit
def _run(x, group, mins, maxs):
    mesh = plsc.VectorSubcoreMesh(core_axis_name="c", subcore_axis_name="s")
    sc_kern = functools.partial(
        pl.kernel,
        mesh=mesh,
        compiler_params=pltpu.CompilerParams(needs_layout_passes=False),
        out_type=jax.ShapeDtypeStruct((_N,), jnp.float32),
        scratch_types=(
            [
                pltpu.VMEM((_L,), jnp.float32),
                pltpu.VMEM((_L,), jnp.float32),
            ]
            + [pltpu.VMEM((_CHUNK,), jnp.float32) for _ in range(_NBUF)]
            + [pltpu.VMEM((_CHUNK,), jnp.int32) for _ in range(_NBUF)]
            + [pltpu.VMEM((_CHUNK,), jnp.float32) for _ in range(_NBUF)]
            + [pltpu.SemaphoreType.DMA for _ in range(_NBUF)]
            + [pltpu.SemaphoreType.DMA for _ in range(_NBUF)]
        ),
    )(_sc_body)
    sc_out = sc_kern(x, group, mins, maxs)

    tc_out = pl.pallas_call(
        _tc_body,
        grid=(_N_TC_BLOCKS,),
        in_specs=[
            pl.BlockSpec((_BLK,), lambda i: (_E_SC // _BLK + i,)),
            pl.BlockSpec((_BLK,), lambda i: (_E_SC // _BLK + i,)),
            pl.BlockSpec((_L,), lambda i: (0,)),
            pl.BlockSpec((_L,), lambda i: (0,)),
        ],
        out_specs=pl.BlockSpec((_BLK,), lambda i: (i,)),
        out_shape=jax.ShapeDtypeStruct((_N - _E_SC,), jnp.float32),
        compiler_params=pltpu.CompilerParams(
            dimension_semantics=("arbitrary",)),
    )(x, group, mins, maxs)

    return lax.dynamic_update_slice(sc_out, tc_out, (_E_SC,))


def kernel(x, group, mins, maxs):
    return _run(x, group, mins, maxs)


# speedup vs baseline: 2.3777x; 1.0960x over previous
import functools

import jax
import jax.numpy as jnp
from jax import lax
from jax.experimental import pallas as pl
from jax.experimental.pallas import tpu as pltpu
from jax.experimental.pallas import tpu_sc as plsc

_EPS = 1e-08
_N = 3276800
_NC = 2
_NS = 16
_NW = _NC * _NS
_PER_W = _N // _NW
_NBUF = 4
_CHUNK = 6400
_NCHUNK = _PER_W // _CHUNK
_L = 16


def _body(x_hbm, g_hbm, mins_hbm, maxs_hbm, out_hbm, scale_v, offs_v, *rest):
    xbufs = rest[0:_NBUF]
    gbufs = rest[_NBUF:2 * _NBUF]
    obufs = rest[2 * _NBUF:3 * _NBUF]
    ld_sems = rest[3 * _NBUF:4 * _NBUF]
    st_sems = rest[4 * _NBUF:5 * _NBUF]

    wid = lax.axis_index("s") * _NC + lax.axis_index("c")
    base = wid * _PER_W

    def start_load(c, b):
        off = base + c * _CHUNK
        pltpu.async_copy(x_hbm.at[pl.ds(off, _CHUNK)], xbufs[b], ld_sems[b])
        pltpu.async_copy(g_hbm.at[pl.ds(off, _CHUNK)], gbufs[b], ld_sems[b])

    def wait_load(b):
        pltpu.make_async_copy(
            x_hbm.at[pl.ds(0, _CHUNK)], xbufs[b], ld_sems[b]).wait()
        pltpu.make_async_copy(
            g_hbm.at[pl.ds(0, _CHUNK)], gbufs[b], ld_sems[b]).wait()

    def wait_store(b):
        pltpu.make_async_copy(
            obufs[b], out_hbm.at[pl.ds(0, _CHUNK)], st_sems[b]).wait()

    def start_store(c, b):
        off = base + c * _CHUNK
        pltpu.async_copy(obufs[b], out_hbm.at[pl.ds(off, _CHUNK)], st_sems[b])

    def compute(b, scale_reg, offs_reg):
        xb, gb, ob = xbufs[b], gbufs[b], obufs[b]

        @plsc.parallel_loop(0, _CHUNK // _L, unroll=8)
        def vec_body(i):
            s = pl.ds(i * _L, _L)
            idx = gb[s] - 1
            sg = jnp.take_along_axis(
                scale_reg, idx, axis=0, mode="promise_in_bounds")
            og = jnp.take_along_axis(
                offs_reg, idx, axis=0, mode="promise_in_bounds")
            ob[s] = xb[s] * sg + og

    for b in range(_NBUF - 1):
        start_load(b, b)

    pltpu.sync_copy(mins_hbm, scale_v)
    pltpu.sync_copy(maxs_hbm, offs_v)
    m = scale_v[...]
    M = offs_v[...]
    scale_reg = (1.0 - 2.0 * _EPS) / (M - m)
    offs_reg = _EPS - m * scale_reg

    @pl.loop(0, _NCHUNK, step=_NBUF)
    def chunk_ring(c):
        for j in range(_NBUF):
            b = j
            cj = c + j

            wait_load(b)

            @pl.when(cj >= _NBUF)
            def _():
                wait_store(b)

            compute(b, scale_reg, offs_reg)
            start_store(cj, b)

            @pl.when(cj + _NBUF - 1 < _NCHUNK)
            def _():
                start_load(cj + _NBUF - 1, (b + _NBUF - 1) % _NBUF)

    for b in range(_NBUF):
        wait_store(b)


@jax.jit
def _run(x, group, mins, maxs):
    mesh = plsc.VectorSubcoreMesh(core_axis_name="c", subcore_axis_name="s")
    kern = functools.partial(
        pl.kernel,
        mesh=mesh,
        compiler_params=pltpu.CompilerParams(needs_layout_passes=False),
        out_type=jax.ShapeDtypeStruct((_N,), jnp.float32),
        scratch_types=(
            [
                pltpu.VMEM((_L,), jnp.float32),
                pltpu.VMEM((_L,), jnp.float32),
            ]
            + [pltpu.VMEM((_CHUNK,), jnp.float32) for _ in range(_NBUF)]
            + [pltpu.VMEM((_CHUNK,), jnp.int32) for _ in range(_NBUF)]
            + [pltpu.VMEM((_CHUNK,), jnp.float32) for _ in range(_NBUF)]
            + [pltpu.SemaphoreType.DMA for _ in range(_NBUF)]
            + [pltpu.SemaphoreType.DMA for _ in range(_NBUF)]
        ),
    )(_body)
    return kern(x, group, mins, maxs)


def kernel(x, group, mins, maxs):
    return _run(x, group, mins, maxs)
